# baseline (speedup 1.0000x reference)
import functools

import jax
import jax.numpy as jnp
from jax import lax
from jax.experimental import pallas as pl
from jax.experimental.pallas import tpu as pltpu

N_VOCAB = 4096
D = 4096
B_TOTAL = 16 * 2048
NSEM = 8
LAG = 32


def _tc_body(idx_ref, table_ref, out_ref, *sems):
    ng = B_TOTAL // NSEM
    lagg = LAG // NSEM

    def issue(i, b):
        r = idx_ref[i]
        pltpu.make_async_copy(
            table_ref.at[pl.ds(r, 1)], out_ref.at[pl.ds(i, 1)],
            sems[b]).start()

    def drain_one(i, b):
        pltpu.make_async_copy(
            table_ref.at[pl.ds(0, 1)], out_ref.at[pl.ds(i, 1)],
            sems[b]).wait()

    def prelude(g, c):
        for b in range(NSEM):
            issue(g * NSEM + b, b)
        return c

    lax.fori_loop(0, lagg, prelude, 0)

    def body(g, c):
        for b in range(NSEM):
            issue(g * NSEM + b, b)
            drain_one((g - lagg) * NSEM + b, b)
        return c

    lax.fori_loop(lagg, ng, body, 0)

    def drain(g, c):
        for b in range(NSEM):
            drain_one(g * NSEM + b, b)
        return c

    lax.fori_loop(ng - lagg, ng, drain, 0)


@jax.jit
def _tc_gather(flat_idx, table):
    grid_spec = pltpu.PrefetchScalarGridSpec(
        num_scalar_prefetch=1,
        grid=(1,),
        in_specs=[pl.BlockSpec(memory_space=pltpu.MemorySpace.HBM)],
        out_specs=pl.BlockSpec(memory_space=pltpu.MemorySpace.HBM),
        scratch_shapes=[pltpu.SemaphoreType.DMA] * NSEM,
    )
    return pl.pallas_call(
        _tc_body,
        grid_spec=grid_spec,
        out_shape=jax.ShapeDtypeStruct((B_TOTAL, D), jnp.float32),
    )(flat_idx, table)


def kernel(indices, table):
    flat = indices.reshape(-1)
    out = _tc_gather(flat, table)
    return out.reshape(indices.shape[0], indices.shape[1], N_VOCAB)


# speedup vs baseline: 41.9700x; 41.9700x over previous
import functools

import jax
import jax.numpy as jnp
from jax import lax
from jax.experimental import pallas as pl
from jax.experimental.pallas import tpu as pltpu
from jax.experimental.pallas import tpu_sc as plsc

N_VOCAB = 4096
D = 4096
B_TOTAL = 16 * 2048
NC = 2
NS = 16
NW = NC * NS
B_PER_W = B_TOTAL // NW
C = 8
N_CH = B_PER_W // C
N_PAIR = N_CH // 2

_mesh = plsc.VectorSubcoreMesh(core_axis_name="c", subcore_axis_name="s")


@functools.partial(
    pl.kernel,
    mesh=_mesh,
    out_type=jax.ShapeDtypeStruct((B_TOTAL, D), jnp.float32),
    scratch_types=[
        pltpu.VMEM((B_PER_W,), jnp.int32),
        pltpu.VMEM((C, D), jnp.float32),
        pltpu.VMEM((C, D), jnp.float32),
        pltpu.SemaphoreType.DMA,
        pltpu.SemaphoreType.DMA,
        pltpu.SemaphoreType.DMA,
        pltpu.SemaphoreType.DMA,
    ],
)
def _gather_kernel(idx_hbm, table_hbm, out_hbm, idx_v, buf0, buf1,
                   g0, g1, o0, o1):
    wid = lax.axis_index("s") * NC + lax.axis_index("c")
    base = wid * B_PER_W
    pltpu.sync_copy(idx_hbm.at[pl.ds(base, B_PER_W)], idx_v)

    bufs = (buf0, buf1)
    gsems = (g0, g1)
    osems = (o0, o1)

    def gather(j, b):
        return pltpu.make_async_copy(
            table_hbm.at[idx_v.at[pl.ds(j * C, C)]], bufs[b], gsems[b])

    def out_copy(j, b):
        return pltpu.make_async_copy(
            bufs[b], out_hbm.at[pl.ds(base + j * C, C)], osems[b])

    gather(0, 0).start()
    gather(1, 1).start()

    def body(jj, carry):
        for b in range(2):
            j = jj * 2 + b
            gather(j, b).wait()
            out_copy(j, b).start()
            out_copy(j, b).wait()
            gather(j + 2, b).start()
        return carry

    lax.fori_loop(0, N_PAIR - 1, body, 0)

    for b in range(2):
        j = (N_PAIR - 1) * 2 + b
        gather(j, b).wait()
        out_copy(j, b).start()
    for b in range(2):
        j = (N_PAIR - 1) * 2 + b
        out_copy(j, b).wait()


def kernel(indices, table):
    flat = indices.reshape(-1)
    out = _gather_kernel(flat, table)
    return out.reshape(indices.shape[0], indices.shape[1], N_VOCAB)
